# baseline (speedup 1.0000x reference)
import functools

import jax
import jax.numpy as jnp
from jax.experimental import pallas as pl
from jax.experimental.pallas import tpu as pltpu

N0, N1, N2 = 50000, 25000, 10000
E0, E1 = 400000, 160000
D = 128


def _rows_blockspec(blk, cols):
    return pl.BlockSpec((blk, cols), lambda i: (i, 0))


def _full_spec(shape):
    return pl.BlockSpec(shape, lambda i: tuple(0 for _ in shape))


def _stage_a(feat0, W_proj, b_proj, W_pool1, b_pool1):
    blk = 1000

    def body(x_ref, wp_ref, bp_ref, wl_ref, bl_ref, h_ref, hp_ref):
        x = x_ref[...]
        h = jnp.dot(x, wp_ref[...], preferred_element_type=jnp.float32) + bp_ref[...]
        h_ref[...] = h
        hp_ref[...] = jnp.maximum(
            jnp.dot(h, wl_ref[...], preferred_element_type=jnp.float32) + bl_ref[...], 0.0)

    return pl.pallas_call(
        body,
        grid=(N0 // blk,),
        in_specs=[
            _rows_blockspec(blk, D),
            _full_spec((D, D)), _full_spec((1, D)),
            _full_spec((D, D)), _full_spec((1, D)),
        ],
        out_specs=[_rows_blockspec(blk, D), _rows_blockspec(blk, D)],
        out_shape=[
            jax.ShapeDtypeStruct((N0, D), jnp.float32),
            jax.ShapeDtypeStruct((N0, D), jnp.float32),
        ],
    )(feat0, W_proj, b_proj.reshape(1, D), W_pool1, b_pool1.reshape(1, D))


def _stage_c(h_dst, agg1, W_self1, W_neigh1, b1, W_pool2, b_pool2):
    blk = 1000
    D2 = 2 * D

    def body(hd_ref, ag_ref, ws_ref, wn_ref, b_ref, wp_ref, bp_ref, h1_ref, hp2_ref):
        h1 = (jnp.dot(hd_ref[...], ws_ref[...], preferred_element_type=jnp.float32)
              + jnp.dot(ag_ref[...], wn_ref[...], preferred_element_type=jnp.float32)
              + b_ref[...])
        h1 = jnp.maximum(h1, 0.0)
        h1_ref[...] = h1
        hp2_ref[...] = jnp.maximum(
            jnp.dot(h1, wp_ref[...], preferred_element_type=jnp.float32) + bp_ref[...], 0.0)

    return pl.pallas_call(
        body,
        grid=(N1 // blk,),
        in_specs=[
            _rows_blockspec(blk, D), _rows_blockspec(blk, D),
            _full_spec((D, D2)), _full_spec((D, D2)), _full_spec((1, D2)),
            _full_spec((D2, D2)), _full_spec((1, D2)),
        ],
        out_specs=[_rows_blockspec(blk, D2), _rows_blockspec(blk, D2)],
        out_shape=[
            jax.ShapeDtypeStruct((N1, D2), jnp.float32),
            jax.ShapeDtypeStruct((N1, D2), jnp.float32),
        ],
    )(h_dst, agg1, W_self1, W_neigh1, b1.reshape(1, D2), W_pool2, b_pool2.reshape(1, D2))


def _stage_e(h_item_dst, h1_dst, agg2, W_self2, W_neigh2, b2):
    blk = 1000
    D2 = 2 * D

    def body(hi_ref, hd_ref, ag_ref, ws_ref, wn_ref, b_ref, out_ref):
        out_ref[...] = (hi_ref[...]
                        + jnp.dot(hd_ref[...], ws_ref[...], preferred_element_type=jnp.float32)
                        + jnp.dot(ag_ref[...], wn_ref[...], preferred_element_type=jnp.float32)
                        + b_ref[...])

    return pl.pallas_call(
        body,
        grid=(N2 // blk,),
        in_specs=[
            _rows_blockspec(blk, D), _rows_blockspec(blk, D2), _rows_blockspec(blk, D2),
            _full_spec((D2, D)), _full_spec((D2, D)), _full_spec((1, D)),
        ],
        out_specs=_rows_blockspec(blk, D),
        out_shape=jax.ShapeDtypeStruct((N2, D), jnp.float32),
    )(h_item_dst, h1_dst, agg2, W_self2, W_neigh2, b2.reshape(1, D))


def kernel(feat0, edge0_src, edge0_dst, edge1_src, edge1_dst, pos_u, pos_v, neg_u, neg_v,
           W_proj, b_proj, W_pool1, b_pool1, W_self1, W_neigh1, b1,
           W_pool2, b_pool2, W_self2, W_neigh2, b2):
    h_item, hp1 = _stage_a(feat0, W_proj, b_proj, W_pool1, b_pool1)

    msg = jnp.take(hp1, edge0_src, axis=0)
    agg1 = jax.ops.segment_max(msg, edge0_dst, num_segments=N1)
    agg1 = jnp.where(jnp.isneginf(agg1), 0.0, agg1)

    h1, hp2 = _stage_c(h_item[:N1], agg1, W_self1, W_neigh1, b1, W_pool2, b_pool2)

    msg2 = jnp.take(hp2, edge1_src, axis=0)
    agg2 = jax.ops.segment_max(msg2, edge1_dst, num_segments=N2)
    agg2 = jnp.where(jnp.isneginf(agg2), 0.0, agg2)

    h = _stage_e(h_item[:N2], h1[:N2], agg2, W_self2, W_neigh2, b2)

    pos_score = jnp.sum(h[pos_u] * h[pos_v], axis=-1)
    neg_score = jnp.sum(h[neg_u] * h[neg_v], axis=-1)
    return (pos_score, neg_score)
